# baseline (speedup 1.0000x reference)
import functools

import jax
import jax.numpy as jnp
from jax import lax
from jax.experimental import pallas as pl
from jax.experimental.pallas import tpu as pltpu
from jax.experimental.pallas import tpu_sc as plsc

N = 1024
D = 2048
E = 8
HID = 512
H2 = 2 * HID
OUT = 2048
SCALE = 12.0
EPS = 1e-5
NORM_EPS = 1e-4

PREP_CHUNK = 256
FIN_CHUNK = 256

_NC = 2
_NS = 16
_NW = _NC * _NS
_LANES = 16
_TOK_W = N // _NW
_ELE_W = _TOK_W * E


def _silu(g):
    return g * jax.nn.sigmoid(g)


def _logits_body(flat_ref, lnpre_ref, rw_ref, logits_ref):
    rw = rw_ref[...]
    wn = jnp.sqrt(jnp.sum(rw * rw, axis=1, keepdims=True))
    wq = rw / jnp.maximum(wn, NORM_EPS)
    flat = flat_ref[...]
    ms = jnp.mean(flat * flat, axis=1, keepdims=True)
    xs = flat * lax.rsqrt(ms + EPS) * lnpre_ref[...]
    xn = jnp.sqrt(jnp.sum(xs * xs, axis=1, keepdims=True))
    xq = xs / jnp.maximum(xn, NORM_EPS)
    logits_ref[...] = jax.lax.dot_general(
        wq, xq, (((1,), (1,)), ((), ())),
        preferred_element_type=jnp.float32) * SCALE


def _sc_router_body(logits_hbm, gates_hbm, lbuf, gbuf, sem):
    wid = lax.axis_index("s") * _NC + lax.axis_index("c")
    base = wid * _TOK_W
    cps = [pltpu.async_copy(logits_hbm.at[t, pl.ds(base, _TOK_W)],
                            lbuf.at[pl.ds(t * _TOK_W, _TOK_W)], sem)
           for t in range(E)]
    for c in cps:
        c.wait()
    for grp in range(_TOK_W // _LANES):
        neg = jnp.full((_LANES,), -1e30, jnp.float32)
        m1 = neg
        m2 = neg
        i1 = jnp.zeros((_LANES,), jnp.int32)
        i2 = jnp.zeros((_LANES,), jnp.int32)
        for t in range(E):
            le = lbuf[pl.ds(t * _TOK_W + grp * _LANES, _LANES)]
            gt1 = le > m1
            gt2 = le > m2
            m2n = jnp.where(gt1, m1, jnp.where(gt2, le, m2))
            i2n = jnp.where(gt1, i1, jnp.where(gt2, t, i2))
            m1 = jnp.where(gt1, le, m1)
            i1 = jnp.where(gt1, t, i1)
            m2, i2 = m2n, i2n
        w1 = 1.0 / (1.0 + jnp.exp(m2 - m1))
        w2 = 1.0 - w1
        zero = jnp.zeros((_LANES,), jnp.float32)
        for t in range(E):
            gbuf[pl.ds(t * _TOK_W + grp * _LANES, _LANES)] = jnp.where(
                i1 == t, w1, jnp.where(i2 == t, w2, zero))
    cps = [pltpu.async_copy(gbuf.at[pl.ds(t * _TOK_W, _TOK_W)],
                            gates_hbm.at[t, pl.ds(base, _TOK_W)], sem)
           for t in range(E)]
    for c in cps:
        c.wait()


@functools.partial(
    pl.kernel,
    mesh=plsc.VectorSubcoreMesh(core_axis_name="c", subcore_axis_name="s"),
    out_type=jax.ShapeDtypeStruct((E, N), jnp.float32),
    scratch_types=[
        pltpu.VMEM((_ELE_W,), jnp.float32),
        pltpu.VMEM((_ELE_W,), jnp.float32),
        pltpu.SemaphoreType.DMA,
    ],
)
def _sc_router(logits_hbm, gates_hbm, lbuf, gbuf, sem):
    _sc_router_body(logits_hbm, gates_hbm, lbuf, gbuf, sem)


def _main_body(flat_ref, lnpre_ref, gates_ref, sw12_ref, sw3_ref,
               ew12_ref, ew3_ref, lnpost_ref, out_ref, xs_ref, g_ref):
    e = pl.program_id(0)
    j = pl.program_id(1)

    @pl.when(jnp.logical_and(e == 0, j == 0))
    def _prep():
        lnpre = lnpre_ref[...]
        for c in range(N // PREP_CHUNK):
            rows = pl.ds(c * PREP_CHUNK, PREP_CHUNK)
            flat = flat_ref[rows, :]
            ms = jnp.mean(flat * flat, axis=1, keepdims=True)
            xs = flat * lax.rsqrt(ms + EPS) * lnpre
            xs_ref[rows, :] = xs.astype(jnp.bfloat16)

        xsb = xs_ref[...]
        gs = jax.lax.dot_general(
            xsb, sw12_ref[pl.ds(0, HID), :].astype(jnp.bfloat16),
            (((1,), (1,)), ((), ())), preferred_element_type=jnp.float32)
        vs = jax.lax.dot_general(
            xsb, sw12_ref[pl.ds(HID, HID), :].astype(jnp.bfloat16),
            (((1,), (1,)), ((), ())), preferred_element_type=jnp.float32)
        actb = (_silu(gs) * vs).astype(jnp.bfloat16)
        for o in range(4):
            cols = pl.ds(o * (OUT // 4), OUT // 4)
            w3b = sw3_ref[cols, :].astype(jnp.bfloat16)
            out_ref[:, cols] = jax.lax.dot_general(
                actb, w3b, (((1,), (1,)), ((), ())),
                preferred_element_type=jnp.float32)

    @pl.when(j == 0)
    def _gate_proj():
        g_ref[...] = jax.lax.dot_general(
            xs_ref[...], ew12_ref[0].astype(jnp.bfloat16),
            (((1,), (1,)), ((), ())), preferred_element_type=jnp.float32)

    @pl.when(j == 1)
    def _value_proj():
        v = jax.lax.dot_general(
            xs_ref[...], ew12_ref[0].astype(jnp.bfloat16),
            (((1,), (1,)), ((), ())), preferred_element_type=jnp.float32)
        lane = jax.lax.broadcasted_iota(jnp.int32, (N, E), 1)
        gate = jnp.sum(jnp.where(lane == e, gates_ref[...], 0.0),
                       axis=1, keepdims=True)
        actb = (_silu(g_ref[...]) * v * gate).astype(jnp.bfloat16)
        for o in range(4):
            cols = pl.ds(o * (OUT // 4), OUT // 4)
            w3b = ew3_ref[0, cols, :].astype(jnp.bfloat16)
            out_ref[:, cols] += jax.lax.dot_general(
                actb, w3b, (((1,), (1,)), ((), ())),
                preferred_element_type=jnp.float32)

        @pl.when(e == E - 1)
        def _fin():
            lnpost = lnpost_ref[...]
            for c in range(N // FIN_CHUNK):
                rows = pl.ds(c * FIN_CHUNK, FIN_CHUNK)
                r = out_ref[rows, :]
                ms = jnp.mean(r * r, axis=1, keepdims=True)
                out_ref[rows, :] = r * lax.rsqrt(ms + EPS) * lnpost


def _full(shape):
    return pl.BlockSpec(shape, lambda *_: (0,) * len(shape))


def _impl(x, ln_pre_w, ln_post_w, router_w, shared_w12, shared_w3,
          expert_w12, expert_w3, interpret):
    flat = x.reshape(N, D)
    lnpre = ln_pre_w.reshape(1, D)
    lnpost = ln_post_w.reshape(1, OUT)

    logits_t = pl.pallas_call(
        _logits_body,
        grid=(N // PREP_CHUNK,),
        in_specs=[
            pl.BlockSpec((PREP_CHUNK, D), lambda c: (c, 0)),
            _full((1, D)),
            _full((E, D)),
        ],
        out_specs=pl.BlockSpec((E, PREP_CHUNK), lambda c: (0, c)),
        out_shape=jax.ShapeDtypeStruct((E, N), jnp.float32),
        compiler_params=pltpu.CompilerParams(
            dimension_semantics=("arbitrary",)),
        interpret=interpret,
    )(flat, lnpre, router_w)

    gates = _sc_router(logits_t).T

    out = pl.pallas_call(
        _main_body,
        grid=(E, 2),
        in_specs=[
            _full((N, D)),
            _full((1, D)),
            _full((N, E)),
            _full((H2, D)),
            _full((OUT, HID)),
            pl.BlockSpec((1, HID, D), lambda e, j: (e, j, 0)),
            pl.BlockSpec((1, OUT, HID), lambda e, j: (e, 0, 0)),
            _full((1, OUT)),
        ],
        out_specs=_full((N, OUT)),
        out_shape=jax.ShapeDtypeStruct((N, OUT), jnp.float32),
        scratch_shapes=[
            pltpu.VMEM((N, D), jnp.bfloat16),
            pltpu.VMEM((N, HID), jnp.float32),
        ],
        compiler_params=pltpu.CompilerParams(
            dimension_semantics=("arbitrary", "arbitrary"),
            vmem_limit_bytes=66912256),
        interpret=interpret,
    )(flat, lnpre, gates, shared_w12, shared_w3,
      expert_w12, expert_w3, lnpost)

    return out.reshape(1, N, OUT)


def kernel(x, ln_pre_w, ln_post_w, router_w, shared_w12, shared_w3,
           expert_w12, expert_w3):
    return _impl(x, ln_pre_w, ln_post_w, router_w, shared_w12, shared_w3,
                 expert_w12, expert_w3, False)


# speedup vs baseline: 1.0274x; 1.0274x over previous
import functools

import jax
import jax.numpy as jnp
from jax import lax
from jax.experimental import pallas as pl
from jax.experimental.pallas import tpu as pltpu
from jax.experimental.pallas import tpu_sc as plsc

N = 1024
D = 2048
E = 8
HID = 512
H2 = 2 * HID
OUT = 2048
SCALE = 12.0
EPS = 1e-5
NORM_EPS = 1e-4

PREP_CHUNK = 256
FIN_CHUNK = 256

_NC = 2
_NS = 16
_NW = _NC * _NS
_LANES = 16
_TOK_W = N // _NW
_ELE_W = _TOK_W * E


def _silu(g):
    return g * jax.nn.sigmoid(g)


def _logits_body(flat_ref, lnpre_ref, rw_ref, xs_ref, logits_ref):
    rw = rw_ref[...]
    wn = jnp.sqrt(jnp.sum(rw * rw, axis=1, keepdims=True))
    wq = rw / jnp.maximum(wn, NORM_EPS)
    flat = flat_ref[...]
    ms = jnp.mean(flat * flat, axis=1, keepdims=True)
    xs = flat * lax.rsqrt(ms + EPS) * lnpre_ref[...]
    xs_ref[...] = xs.astype(jnp.bfloat16)
    xn = jnp.sqrt(jnp.sum(xs * xs, axis=1, keepdims=True))
    xq = xs / jnp.maximum(xn, NORM_EPS)
    logits_ref[...] = jax.lax.dot_general(
        wq, xq, (((1,), (1,)), ((), ())),
        preferred_element_type=jnp.float32) * SCALE


def _sc_router_body(logits_hbm, gates_hbm, lbuf, gbuf, sem):
    wid = lax.axis_index("s") * _NC + lax.axis_index("c")
    base = wid * _TOK_W
    cps = [pltpu.async_copy(logits_hbm.at[t, pl.ds(base, _TOK_W)],
                            lbuf.at[pl.ds(t * _TOK_W, _TOK_W)], sem)
           for t in range(E)]
    for c in cps:
        c.wait()
    for grp in range(_TOK_W // _LANES):
        neg = jnp.full((_LANES,), -1e30, jnp.float32)
        m1 = neg
        m2 = neg
        i1 = jnp.zeros((_LANES,), jnp.int32)
        i2 = jnp.zeros((_LANES,), jnp.int32)
        for t in range(E):
            le = lbuf[pl.ds(t * _TOK_W + grp * _LANES, _LANES)]
            gt1 = le > m1
            gt2 = le > m2
            m2n = jnp.where(gt1, m1, jnp.where(gt2, le, m2))
            i2n = jnp.where(gt1, i1, jnp.where(gt2, t, i2))
            m1 = jnp.where(gt1, le, m1)
            i1 = jnp.where(gt1, t, i1)
            m2, i2 = m2n, i2n
        w1 = 1.0 / (1.0 + jnp.exp(m2 - m1))
        w2 = 1.0 - w1
        zero = jnp.zeros((_LANES,), jnp.float32)
        for t in range(E):
            gbuf[pl.ds(t * _TOK_W + grp * _LANES, _LANES)] = jnp.where(
                i1 == t, w1, jnp.where(i2 == t, w2, zero))
    cps = [pltpu.async_copy(gbuf.at[pl.ds(t * _TOK_W, _TOK_W)],
                            gates_hbm.at[t, pl.ds(base, _TOK_W)], sem)
           for t in range(E)]
    for c in cps:
        c.wait()


@functools.partial(
    pl.kernel,
    mesh=plsc.VectorSubcoreMesh(core_axis_name="c", subcore_axis_name="s"),
    out_type=jax.ShapeDtypeStruct((E, N), jnp.float32),
    scratch_types=[
        pltpu.VMEM((_ELE_W,), jnp.float32),
        pltpu.VMEM((_ELE_W,), jnp.float32),
        pltpu.SemaphoreType.DMA,
    ],
)
def _sc_router(logits_hbm, gates_hbm, lbuf, gbuf, sem):
    _sc_router_body(logits_hbm, gates_hbm, lbuf, gbuf, sem)


def _main_body(xs_ref, gatest_ref, sw12_ref, sw3_ref,
               ew12_ref, ew3_ref, lnpost_ref, out_ref, gates_ref, g_ref):
    e = pl.program_id(0)
    j = pl.program_id(1)

    @pl.when(jnp.logical_and(e == 0, j == 0))
    def _prep():
        gates_ref[...] = gatest_ref[...].T

        xsb = xs_ref[...]
        gs = jax.lax.dot_general(
            xsb, sw12_ref[pl.ds(0, HID), :].astype(jnp.bfloat16),
            (((1,), (1,)), ((), ())), preferred_element_type=jnp.float32)
        vs = jax.lax.dot_general(
            xsb, sw12_ref[pl.ds(HID, HID), :].astype(jnp.bfloat16),
            (((1,), (1,)), ((), ())), preferred_element_type=jnp.float32)
        actb = (_silu(gs) * vs).astype(jnp.bfloat16)
        for o in range(4):
            cols = pl.ds(o * (OUT // 4), OUT // 4)
            w3b = sw3_ref[cols, :].astype(jnp.bfloat16)
            out_ref[:, cols] = jax.lax.dot_general(
                actb, w3b, (((1,), (1,)), ((), ())),
                preferred_element_type=jnp.float32)

    @pl.when(j == 0)
    def _gate_proj():
        g_ref[...] = jax.lax.dot_general(
            xs_ref[...], ew12_ref[0].astype(jnp.bfloat16),
            (((1,), (1,)), ((), ())), preferred_element_type=jnp.float32)

    @pl.when(j == 1)
    def _value_proj():
        v = jax.lax.dot_general(
            xs_ref[...], ew12_ref[0].astype(jnp.bfloat16),
            (((1,), (1,)), ((), ())), preferred_element_type=jnp.float32)
        lane = jax.lax.broadcasted_iota(jnp.int32, (N, E), 1)
        gate = jnp.sum(jnp.where(lane == e, gates_ref[...], 0.0),
                       axis=1, keepdims=True)
        actb = (_silu(g_ref[...]) * v * gate).astype(jnp.bfloat16)
        for o in range(4):
            cols = pl.ds(o * (OUT // 4), OUT // 4)
            w3b = ew3_ref[0, cols, :].astype(jnp.bfloat16)
            out_ref[:, cols] += jax.lax.dot_general(
                actb, w3b, (((1,), (1,)), ((), ())),
                preferred_element_type=jnp.float32)

        @pl.when(e == E - 1)
        def _fin():
            lnpost = lnpost_ref[...]
            for c in range(N // FIN_CHUNK):
                rows = pl.ds(c * FIN_CHUNK, FIN_CHUNK)
                r = out_ref[rows, :]
                ms = jnp.mean(r * r, axis=1, keepdims=True)
                out_ref[rows, :] = r * lax.rsqrt(ms + EPS) * lnpost


def _full(shape):
    return pl.BlockSpec(shape, lambda *_: (0,) * len(shape))


def _impl(x, ln_pre_w, ln_post_w, router_w, shared_w12, shared_w3,
          expert_w12, expert_w3, interpret):
    flat = x.reshape(N, D)
    lnpre = ln_pre_w.reshape(1, D)
    lnpost = ln_post_w.reshape(1, OUT)

    xs, logits_t = pl.pallas_call(
        _logits_body,
        grid=(N // PREP_CHUNK,),
        in_specs=[
            pl.BlockSpec((PREP_CHUNK, D), lambda c: (c, 0)),
            _full((1, D)),
            _full((E, D)),
        ],
        out_specs=(
            pl.BlockSpec((PREP_CHUNK, D), lambda c: (c, 0)),
            pl.BlockSpec((E, PREP_CHUNK), lambda c: (0, c)),
        ),
        out_shape=(
            jax.ShapeDtypeStruct((N, D), jnp.bfloat16),
            jax.ShapeDtypeStruct((E, N), jnp.float32),
        ),
        compiler_params=pltpu.CompilerParams(
            dimension_semantics=("arbitrary",)),
        interpret=interpret,
    )(flat, lnpre, router_w)

    gates_t = _sc_router(logits_t)

    out = pl.pallas_call(
        _main_body,
        grid=(E, 2),
        in_specs=[
            _full((N, D)),
            _full((E, N)),
            _full((H2, D)),
            _full((OUT, HID)),
            pl.BlockSpec((1, HID, D), lambda e, j: (e, j, 0)),
            pl.BlockSpec((1, OUT, HID), lambda e, j: (e, 0, 0)),
            _full((1, OUT)),
        ],
        out_specs=_full((N, OUT)),
        out_shape=jax.ShapeDtypeStruct((N, OUT), jnp.float32),
        scratch_shapes=[
            pltpu.VMEM((N, E), jnp.float32),
            pltpu.VMEM((N, HID), jnp.float32),
        ],
        compiler_params=pltpu.CompilerParams(
            dimension_semantics=("arbitrary", "arbitrary"),
            vmem_limit_bytes=66912256),
        interpret=interpret,
    )(xs, gates_t, shared_w12, shared_w3,
      expert_w12, expert_w3, lnpost)

    return out.reshape(1, N, OUT)


def kernel(x, ln_pre_w, ln_post_w, router_w, shared_w12, shared_w3,
           expert_w12, expert_w3):
    return _impl(x, ln_pre_w, ln_post_w, router_w, shared_w12, shared_w3,
                 expert_w12, expert_w3, False)
